# Initial kernel scaffold; baseline (speedup 1.0000x reference)
import jax
import jax.numpy as jnp
from jax.experimental import pallas as pl


def kernel(x, edge_index, batch, Wr1, Wn1, b1, pw1, Wr2, Wn2, b2, pw2, fcW, fcb):
    raise NotImplementedError("write your pallas kernel here")


# speedup vs baseline: 2.0619x; 2.0619x over previous
import functools
import math

import jax
import jax.numpy as jnp
from jax import lax
from jax.experimental import pallas as pl
from jax.experimental.pallas import tpu as pltpu

_N, _E = 10000, 160000
_NPAD = 10240
_BR = 1024
_INTERPRET = False


def _mm_body(x_ref, a_ref, wr_ref, wn_ref, b_ref, w_ref, sp_ref, thrp_ref,
             h_ref, s_ref):
    h = jnp.dot(x_ref[...], wr_ref[...], preferred_element_type=jnp.float32)
    h = h + jnp.dot(a_ref[...], wn_ref[...], preferred_element_type=jnp.float32)
    h = jnp.maximum(h + b_ref[...], 0.0)
    h_ref[...] = h
    wv = w_ref[...]
    inv = lax.rsqrt(jnp.sum(wv * wv))
    s = jnp.tanh(jnp.dot(h, wv, preferred_element_type=jnp.float32) * inv)
    bid = pl.program_id(0)
    row = bid * _BR + lax.broadcasted_iota(jnp.int32, s.shape, 0)
    keep = (sp_ref[...] >= thrp_ref[0, 0]) & (row < _N)
    s_ref[...] = jnp.where(keep, s, -2.0)


def _mm(x, agg, wr, wn, b, w, s_prev, thr_prev):
    d = x.shape[1]
    grid = _NPAD // _BR
    return pl.pallas_call(
        _mm_body,
        grid=(grid,),
        in_specs=[
            pl.BlockSpec((_BR, d), lambda i: (i, 0)),
            pl.BlockSpec((_BR, d), lambda i: (i, 0)),
            pl.BlockSpec((d, 512), lambda i: (0, 0)),
            pl.BlockSpec((d, 512), lambda i: (0, 0)),
            pl.BlockSpec((1, 512), lambda i: (0, 0)),
            pl.BlockSpec((512, 1), lambda i: (0, 0)),
            pl.BlockSpec((_BR, 1), lambda i: (i, 0)),
            pl.BlockSpec((1, 1), lambda i: (0, 0), memory_space=pltpu.SMEM),
        ],
        out_specs=[
            pl.BlockSpec((_BR, 512), lambda i: (i, 0)),
            pl.BlockSpec((_BR, 1), lambda i: (i, 0)),
        ],
        out_shape=[
            jax.ShapeDtypeStruct((_NPAD, 512), jnp.float32),
            jax.ShapeDtypeStruct((_NPAD, 1), jnp.float32),
        ],
        interpret=_INTERPRET,
    )(x, agg, wr, wn, b.reshape(1, 512), w.reshape(512, 1), s_prev, thr_prev)


def _bisect(s2d, k):
    kf = jnp.float32(k)

    def body(_, lohi):
        lo, hi = lohi
        mid = (lo + hi) * 0.5
        cnt = jnp.sum((s2d >= mid).astype(jnp.float32))
        pred = cnt >= kf
        return jnp.where(pred, mid, lo), jnp.where(pred, hi, mid)

    lo, _ = lax.fori_loop(0, 64, body, (jnp.float32(-1.5), jnp.float32(1.0)))
    return lo


def _thresh_body(k, s_ref, thr_ref):
    s2d = jnp.reshape(s_ref[...], (_NPAD // 128, 128))
    thr_ref[0, 0] = _bisect(s2d, k)


def _thresh(s, k):
    return pl.pallas_call(
        functools.partial(_thresh_body, k),
        in_specs=[pl.BlockSpec((_NPAD, 1), lambda: (0, 0))],
        out_specs=pl.BlockSpec(memory_space=pltpu.SMEM),
        out_shape=jax.ShapeDtypeStruct((1, 1), jnp.float32),
        interpret=_INTERPRET,
    )(s)


def _scale_body(h_ref, s_ref, thr_ref, o_ref):
    s = s_ref[...]
    w = jnp.where(s >= thr_ref[0, 0], s, 0.0)
    o_ref[...] = h_ref[...] * w


def _scale(h, s, thr):
    grid = _NPAD // _BR
    return pl.pallas_call(
        _scale_body,
        grid=(grid,),
        in_specs=[
            pl.BlockSpec((_BR, 512), lambda i: (i, 0)),
            pl.BlockSpec((_BR, 1), lambda i: (i, 0)),
            pl.BlockSpec((1, 1), lambda i: (0, 0), memory_space=pltpu.SMEM),
        ],
        out_specs=pl.BlockSpec((_BR, 512), lambda i: (i, 0)),
        out_shape=jax.ShapeDtypeStruct((_NPAD, 512), jnp.float32),
        interpret=_INTERPRET,
    )(h, s, thr)


def _final_body(k, h_ref, s_ref, fcw_ref, fcb_ref, o_ref):
    s = s_ref[...]
    thr = _bisect(jnp.reshape(s, (_NPAD // 128, 128)), k)
    w = jnp.where(s >= thr, s, 0.0)
    pooled = jnp.sum(h_ref[...] * w, axis=0, keepdims=True) / jnp.float32(k)
    o_ref[...] = (jnp.dot(pooled, fcw_ref[...], preferred_element_type=jnp.float32)
                  + fcb_ref[...])


def _final(h, s, fcw, fcb, k):
    return pl.pallas_call(
        functools.partial(_final_body, k),
        in_specs=[
            pl.BlockSpec((_NPAD, 512), lambda: (0, 0)),
            pl.BlockSpec((_NPAD, 1), lambda: (0, 0)),
            pl.BlockSpec((512, 128), lambda: (0, 0)),
            pl.BlockSpec((1, 128), lambda: (0, 0)),
        ],
        out_specs=pl.BlockSpec((1, 128), lambda: (0, 0)),
        out_shape=jax.ShapeDtypeStruct((1, 128), jnp.float32),
        interpret=_INTERPRET,
    )(h, s, fcw, fcb.reshape(1, 128))


def _scatter_rows(x, src, dst):
    return jnp.zeros((_NPAD, x.shape[1]), x.dtype).at[dst].add(x[src])


def kernel(x, edge_index, batch, Wr1, Wn1, b1, pw1, Wr2, Wn2, b2, pw2, fcW, fcb):
    del batch
    src, dst = edge_index[0], edge_index[1]
    k1 = int(math.ceil(0.8 * _N))
    k2 = int(math.ceil(0.8 * k1))

    xp = jnp.pad(x, ((0, _NPAD - _N), (0, 0)))
    s_zero = jnp.zeros((_NPAD, 1), jnp.float32)
    thr_lo = jnp.full((1, 1), -1.5, jnp.float32)

    agg1 = _scatter_rows(xp, src, dst)
    h1, s1 = _mm(xp, agg1, Wr1, Wn1, b1, pw1, s_zero, thr_lo)
    thr1 = _thresh(s1, k1)
    x2 = _scale(h1, s1, thr1)
    agg2 = _scatter_rows(x2, src, dst)
    h2, s2 = _mm(x2, agg2, Wr2, Wn2, b2, pw2, s1, thr1)
    return _final(h2, s2, fcW, fcb, k2)


# speedup vs baseline: 3.0033x; 1.4565x over previous
import functools
import math

import jax
import jax.numpy as jnp
from jax import lax
from jax.experimental import pallas as pl
from jax.experimental.pallas import tpu as pltpu
from jax.experimental.pallas import tpu_sc as plsc

_N, _E = 10000, 160000
_NPAD = 10240
_BR = 1024
_INTERPRET = False


def _mmw_body(x_ref, wn_ref, wr_ref, v_ref, u_ref):
    v_ref[...] = jnp.dot(x_ref[...], wn_ref[...],
                         preferred_element_type=jnp.float32)
    u_ref[...] = jnp.dot(x_ref[...], wr_ref[...],
                         preferred_element_type=jnp.float32)


def _mmw(x, wn, wr):
    d = x.shape[1]
    return pl.pallas_call(
        _mmw_body,
        grid=(_NPAD // _BR,),
        in_specs=[
            pl.BlockSpec((_BR, d), lambda i: (i, 0)),
            pl.BlockSpec((d, 512), lambda i: (0, 0)),
            pl.BlockSpec((d, 512), lambda i: (0, 0)),
        ],
        out_specs=[
            pl.BlockSpec((_BR, 512), lambda i: (i, 0)),
            pl.BlockSpec((_BR, 512), lambda i: (i, 0)),
        ],
        out_shape=[
            jax.ShapeDtypeStruct((_NPAD, 512), jnp.float32),
            jax.ShapeDtypeStruct((_NPAD, 512), jnp.float32),
        ],
        interpret=_INTERPRET,
    )(x, wn, wr)


def _mmb_body(u_ref, a_ref, b_ref, w_ref, nrm_ref, mp_ref, h_ref, s_ref):
    h = jnp.maximum((u_ref[...] + a_ref[...]) + b_ref[...], 0.0)
    h_ref[...] = h
    z = jnp.dot(h, w_ref[...], preferred_element_type=jnp.float32)
    s = jnp.tanh(z / nrm_ref[0, 0])
    bid = pl.program_id(0)
    row = bid * _BR + lax.broadcasted_iota(jnp.int32, s.shape, 0)
    keep = (mp_ref[...] > 0.0) & (row < _N)
    s_ref[...] = jnp.where(keep, s, -2.0)


def _mmb(u, agg, b, w, nrm, mprev):
    return pl.pallas_call(
        _mmb_body,
        grid=(_NPAD // _BR,),
        in_specs=[
            pl.BlockSpec((_BR, 512), lambda i: (i, 0)),
            pl.BlockSpec((_BR, 512), lambda i: (i, 0)),
            pl.BlockSpec((1, 512), lambda i: (0, 0)),
            pl.BlockSpec((512, 1), lambda i: (0, 0)),
            pl.BlockSpec((1, 1), lambda i: (0, 0), memory_space=pltpu.SMEM),
            pl.BlockSpec((_BR, 1), lambda i: (i, 0)),
        ],
        out_specs=[
            pl.BlockSpec((_BR, 512), lambda i: (i, 0)),
            pl.BlockSpec((_BR, 1), lambda i: (i, 0)),
        ],
        out_shape=[
            jax.ShapeDtypeStruct((_NPAD, 512), jnp.float32),
            jax.ShapeDtypeStruct((_NPAD, 1), jnp.float32),
        ],
        interpret=_INTERPRET,
    )(u, agg, b.reshape(1, 512), w.reshape(512, 1), nrm, mprev)


def _bisect(s2d, k):
    kf = jnp.float32(k)

    def body(_, lohi):
        lo, hi = lohi
        mid = (lo + hi) * 0.5
        cnt = jnp.sum((s2d >= mid).astype(jnp.float32))
        pred = cnt >= kf
        return jnp.where(pred, mid, lo), jnp.where(pred, hi, mid)

    lo, _ = lax.fori_loop(0, 64, body, (jnp.float32(-1.5), jnp.float32(1.0)))
    return lo


def _select(s2d, k):
    vk = _bisect(s2d, k)
    gt = s2d > vk
    eq = (s2d == vk).astype(jnp.float32)
    cnt_gt = jnp.sum(gt.astype(jnp.float32))
    need = jnp.float32(k) - cnt_gt
    nr, nc = s2d.shape
    ra = lax.broadcasted_iota(jnp.int32, (nc, nc), 0)
    rb = lax.broadcasted_iota(jnp.int32, (nc, nc), 1)
    m_incl = (ra <= rb).astype(jnp.float32)
    cum_in_row = jnp.dot(eq, m_incl, preferred_element_type=jnp.float32)
    ex_in_row = cum_in_row - eq
    rowtot = jnp.sum(eq, axis=1, keepdims=True)
    qa = lax.broadcasted_iota(jnp.int32, (nr, nr), 0)
    qb = lax.broadcasted_iota(jnp.int32, (nr, nr), 1)
    m_strict = (qb < qa).astype(jnp.float32)
    rowpre = jnp.dot(m_strict, rowtot, preferred_element_type=jnp.float32)
    rank = ex_in_row + rowpre
    return gt | ((eq > 0.0) & (rank < need))


def _sel_body(k, s_ref, w_ref, m_ref):
    s2d = jnp.reshape(s_ref[...], (_NPAD // 128, 128))
    sel = _select(s2d, k)
    w2d = jnp.where(sel, s2d, 0.0)
    w_ref[...] = jnp.reshape(w2d, (_NPAD, 1))
    m_ref[...] = jnp.reshape(sel.astype(jnp.float32), (_NPAD, 1))


def _sel(s, k):
    return pl.pallas_call(
        functools.partial(_sel_body, k),
        in_specs=[pl.BlockSpec((_NPAD, 1), lambda: (0, 0))],
        out_specs=[
            pl.BlockSpec((_NPAD, 1), lambda: (0, 0)),
            pl.BlockSpec((_NPAD, 1), lambda: (0, 0)),
        ],
        out_shape=[
            jax.ShapeDtypeStruct((_NPAD, 1), jnp.float32),
            jax.ShapeDtypeStruct((_NPAD, 1), jnp.float32),
        ],
        interpret=_INTERPRET,
    )(s)


def _scale_body(h_ref, w_ref, o_ref):
    o_ref[...] = h_ref[...] * w_ref[...]


def _scale(h, w):
    return pl.pallas_call(
        _scale_body,
        grid=(_NPAD // _BR,),
        in_specs=[
            pl.BlockSpec((_BR, 512), lambda i: (i, 0)),
            pl.BlockSpec((_BR, 1), lambda i: (i, 0)),
        ],
        out_specs=pl.BlockSpec((_BR, 512), lambda i: (i, 0)),
        out_shape=jax.ShapeDtypeStruct((_NPAD, 512), jnp.float32),
        interpret=_INTERPRET,
    )(h, w)


def _final_body(k, h_ref, s_ref, fcw_ref, fcb_ref, o_ref):
    s2d = jnp.reshape(s_ref[...], (_NPAD // 128, 128))
    sel = _select(s2d, k)
    w = jnp.reshape(jnp.where(sel, s2d, 0.0), (_NPAD, 1))
    pooled = jnp.sum(h_ref[...] * w, axis=0, keepdims=True) / jnp.float32(k)
    o_ref[...] = (jnp.dot(pooled, fcw_ref[...],
                          preferred_element_type=jnp.float32) + fcb_ref[...])


def _final(h, s, fcw, fcb, k):
    return pl.pallas_call(
        functools.partial(_final_body, k),
        in_specs=[
            pl.BlockSpec((_NPAD, 512), lambda: (0, 0)),
            pl.BlockSpec((_NPAD, 1), lambda: (0, 0)),
            pl.BlockSpec((512, 128), lambda: (0, 0)),
            pl.BlockSpec((1, 128), lambda: (0, 0)),
        ],
        out_specs=pl.BlockSpec((1, 128), lambda: (0, 0)),
        out_shape=jax.ShapeDtypeStruct((1, 128), jnp.float32),
        interpret=_INTERPRET,
    )(h, s, fcw, fcb.reshape(1, 128))


_NT = 32
_RT = _NPAD // _NT
_CAP = 12288
_CAPP = _CAP + 32
_EB = 4000
_CW = 256
_GB = 32

_sc_mesh = plsc.VectorSubcoreMesh(core_axis_name="c", subcore_axis_name="s")


def _bin_pass(src_hbm, dst_hbm, ssrc, sdst, listbuf, cbuf, tid):
    iota16 = lax.iota(jnp.int32, 16)
    base = tid * _RT

    def stage_body(st, cnt):
        pltpu.sync_copy(src_hbm.at[pl.ds(st * _EB, _EB)], ssrc)
        pltpu.sync_copy(dst_hbm.at[pl.ds(st * _EB, _EB)], sdst)

        def vec_body(v, cnt):
            dd = sdst[pl.ds(v * 16, 16)]
            ss = ssrc[pl.ds(v * 16, 16)]
            rel = dd - base
            m = (rel >= 0) & (rel < _RT)
            csum = jnp.cumsum(m.astype(jnp.int32))
            pos = cnt + csum - 1
            m = m & (pos < _CAP)
            pack = rel * 16384 + ss
            plsc.store_scatter(listbuf, [pos], pack, mask=m)
            return cnt + csum[15]

        return lax.fori_loop(0, _EB // 16, vec_body, cnt)

    cnt = lax.fori_loop(0, _E // _EB, stage_body, jnp.int32(0))
    nbat = (cnt + 31) >> 5
    for v in range(2):
        pos = cnt + v * 16 + iota16
        m = pos < nbat * 32
        pack = 10224 + iota16
        plsc.store_scatter(listbuf, [pos], pack, mask=m)
    cbuf[pl.ds(0, 16)] = jnp.full((16,), nbat, jnp.int32)
    return nbat


def _acc_pass(x_hbm, out_hbm, zeros_hbm, listbuf, acc, rowsbufs, gbufs, sems,
              nbat, tid, coff):
    pltpu.sync_copy(zeros_hbm, acc)

    def prep(b, k):
        for v in range(_GB // 16):
            e16 = listbuf[pl.ds(b * _GB + v * 16, 16)]
            gbufs[k][pl.ds(v * 16, 16)] = e16 & 16383
        return pltpu.async_copy(
            x_hbm.at[gbufs[k], pl.ds(coff, _CW)], rowsbufs[k], sems[k])

    @pl.when(nbat > 0)
    def _():
        prep(0, 0)

    @pl.when(nbat > 1)
    def _():
        prep(1, 1)

    def grp_body(g, carry):
        for k in range(2):
            b = g * 2 + k

            @pl.when(b < nbat)
            def _():
                pltpu.make_async_copy(
                    x_hbm.at[gbufs[k], pl.ds(coff, _CW)], rowsbufs[k],
                    sems[k]).wait()
                rows = rowsbufs[k]
                rels = []
                for v in range(_GB // 16):
                    e16 = listbuf[pl.ds(b * _GB + v * 16, 16)]
                    rels.append(lax.shift_right_logical(e16, 14))
                for j in range(_GB):
                    rel = rels[j // 16][j % 16]
                    for ci in range(_CW // 16):
                        sl = pl.ds(ci * 16, 16)
                        plsc.addupdate(acc.at[rel, sl], rows[j, sl])

                @pl.when(b + 2 < nbat)
                def _():
                    prep(b + 2, k)
        return carry

    lax.fori_loop(0, (nbat + 1) >> 1, grp_body, 0)
    pltpu.sync_copy(acc, out_hbm.at[pl.ds(tid * _RT, _RT), pl.ds(coff, _CW)])


@functools.partial(
    pl.kernel,
    out_type=[
        jax.ShapeDtypeStruct((_NPAD, 512), jnp.float32),
        jax.ShapeDtypeStruct((_NT, _CAPP), jnp.int32),
        jax.ShapeDtypeStruct((_NT, 16), jnp.int32),
    ],
    mesh=_sc_mesh,
    compiler_params=pltpu.CompilerParams(needs_layout_passes=False),
    scratch_types=[
        pltpu.VMEM((_EB,), jnp.int32),
        pltpu.VMEM((_EB,), jnp.int32),
        pltpu.VMEM((_CAPP,), jnp.int32),
        pltpu.VMEM((16,), jnp.int32),
        pltpu.VMEM((_RT, _CW), jnp.float32),
        pltpu.VMEM((_GB, _CW), jnp.float32),
        pltpu.VMEM((_GB, _CW), jnp.float32),
        pltpu.VMEM((_GB,), jnp.int32),
        pltpu.VMEM((_GB,), jnp.int32),
        pltpu.SemaphoreType.DMA,
        pltpu.SemaphoreType.DMA,
    ],
)
def _sc_agg_a(x_hbm, src_hbm, dst_hbm, zeros_hbm,
              agg_hbm, lists_hbm, counts_hbm,
              ssrc, sdst, listbuf, cbuf, acc, rows0, rows1, g0, g1, s0, s1):
    tid = lax.axis_index("c") * 16 + lax.axis_index("s")
    nbat = _bin_pass(src_hbm, dst_hbm, ssrc, sdst, listbuf, cbuf, tid)
    pltpu.sync_copy(cbuf, counts_hbm.at[tid])
    pltpu.sync_copy(listbuf, lists_hbm.at[tid])
    for coff in (0, 256):
        _acc_pass(x_hbm, agg_hbm, zeros_hbm, listbuf, acc, (rows0, rows1),
                  (g0, g1), (s0, s1), nbat, tid, coff)


@functools.partial(
    pl.kernel,
    out_type=jax.ShapeDtypeStruct((_NPAD, 512), jnp.float32),
    mesh=_sc_mesh,
    compiler_params=pltpu.CompilerParams(needs_layout_passes=False),
    scratch_types=[
        pltpu.VMEM((_CAPP,), jnp.int32),
        pltpu.VMEM((16,), jnp.int32),
        pltpu.VMEM((_RT, _CW), jnp.float32),
        pltpu.VMEM((_GB, _CW), jnp.float32),
        pltpu.VMEM((_GB, _CW), jnp.float32),
        pltpu.VMEM((_GB,), jnp.int32),
        pltpu.VMEM((_GB,), jnp.int32),
        pltpu.SemaphoreType.DMA,
        pltpu.SemaphoreType.DMA,
    ],
)
def _sc_agg_b(x_hbm, lists_hbm, counts_hbm, zeros_hbm, agg_hbm,
              listbuf, cbuf, acc, rows0, rows1, g0, g1, s0, s1):
    tid = lax.axis_index("c") * 16 + lax.axis_index("s")
    pltpu.sync_copy(lists_hbm.at[tid], listbuf)
    pltpu.sync_copy(counts_hbm.at[tid], cbuf)
    nbat = cbuf[pl.ds(0, 16)][0]
    for coff in (0, 256):
        _acc_pass(x_hbm, agg_hbm, zeros_hbm, listbuf, acc, (rows0, rows1),
                  (g0, g1), (s0, s1), nbat, tid, coff)


def kernel(x, edge_index, batch, Wr1, Wn1, b1, pw1, Wr2, Wn2, b2, pw2, fcW, fcb):
    del batch
    src, dst = edge_index[0], edge_index[1]
    k1 = int(math.ceil(0.8 * _N))
    k2 = int(math.ceil(0.8 * k1))
    nrm1 = jnp.linalg.norm(pw1).reshape(1, 1)
    nrm2 = jnp.linalg.norm(pw2).reshape(1, 1)

    xp = jnp.pad(x, ((0, _NPAD - _N), (0, 0)))
    ones = jnp.ones((_NPAD, 1), jnp.float32)
    zeros = jnp.zeros((_RT, _CW), jnp.float32)

    v1, u1 = _mmw(xp, Wn1, Wr1)
    agg1, lists, counts = _sc_agg_a(v1, src, dst, zeros)
    h1, s1 = _mmb(u1, agg1, b1, pw1, nrm1, ones)
    w1, m1 = _sel(s1, k1)
    x2 = _scale(h1, w1)
    v2, u2 = _mmw(x2, Wn2, Wr2)
    agg2 = _sc_agg_b(v2, lists, counts, zeros)
    h2, s2 = _mmb(u2, agg2, b2, pw2, nrm2, m1)
    return _final(h2, s2, fcW, fcb, k2)


# speedup vs baseline: 5.0033x; 1.6659x over previous
import functools
import math

import jax
import jax.numpy as jnp
from jax import lax
from jax.experimental import pallas as pl
from jax.experimental.pallas import tpu as pltpu
from jax.experimental.pallas import tpu_sc as plsc

_N, _E = 10000, 160000
_NPAD = 10240
_BR = 1024
_INTERPRET = False


def _mmw_body(x_ref, wn_ref, wr_ref, v_ref, u_ref):
    v_ref[...] = jnp.dot(x_ref[...], wn_ref[...],
                         preferred_element_type=jnp.float32)
    u_ref[...] = jnp.dot(x_ref[...], wr_ref[...],
                         preferred_element_type=jnp.float32)


def _mmw(x, wn, wr):
    d = x.shape[1]
    return pl.pallas_call(
        _mmw_body,
        grid=(_NPAD // _BR,),
        in_specs=[
            pl.BlockSpec((_BR, d), lambda i: (i, 0)),
            pl.BlockSpec((d, 512), lambda i: (0, 0)),
            pl.BlockSpec((d, 512), lambda i: (0, 0)),
        ],
        out_specs=[
            pl.BlockSpec((_BR, 512), lambda i: (i, 0)),
            pl.BlockSpec((_BR, 512), lambda i: (i, 0)),
        ],
        out_shape=[
            jax.ShapeDtypeStruct((_NPAD, 512), jnp.float32),
            jax.ShapeDtypeStruct((_NPAD, 512), jnp.float32),
        ],
        interpret=_INTERPRET,
    )(x, wn, wr)


def _mmb_body(u_ref, a_ref, b_ref, w_ref, nrm_ref, mp_ref, h_ref, s_ref):
    h = jnp.maximum((u_ref[...] + a_ref[...]) + b_ref[...], 0.0)
    h_ref[...] = h
    z = jnp.dot(h, w_ref[...], preferred_element_type=jnp.float32)
    s = jnp.tanh(z / nrm_ref[0, 0])
    bid = pl.program_id(0)
    row = bid * _BR + lax.broadcasted_iota(jnp.int32, s.shape, 0)
    keep = (mp_ref[...] > 0.0) & (row < _N)
    s_ref[...] = jnp.where(keep, s, -2.0)


def _mmb(u, agg, b, w, nrm, mprev):
    return pl.pallas_call(
        _mmb_body,
        grid=(_NPAD // _BR,),
        in_specs=[
            pl.BlockSpec((_BR, 512), lambda i: (i, 0)),
            pl.BlockSpec((_BR, 512), lambda i: (i, 0)),
            pl.BlockSpec((1, 512), lambda i: (0, 0)),
            pl.BlockSpec((512, 1), lambda i: (0, 0)),
            pl.BlockSpec((1, 1), lambda i: (0, 0), memory_space=pltpu.SMEM),
            pl.BlockSpec((_BR, 1), lambda i: (i, 0)),
        ],
        out_specs=[
            pl.BlockSpec((_BR, 512), lambda i: (i, 0)),
            pl.BlockSpec((_BR, 1), lambda i: (i, 0)),
        ],
        out_shape=[
            jax.ShapeDtypeStruct((_NPAD, 512), jnp.float32),
            jax.ShapeDtypeStruct((_NPAD, 1), jnp.float32),
        ],
        interpret=_INTERPRET,
    )(u, agg, b.reshape(1, 512), w.reshape(512, 1), nrm, mprev)


def _bisect(s2d, k):
    kf = jnp.float32(k)

    def body(_, lohi):
        lo, hi = lohi
        mid = (lo + hi) * 0.5
        cnt = jnp.sum((s2d >= mid).astype(jnp.float32))
        pred = cnt >= kf
        return jnp.where(pred, mid, lo), jnp.where(pred, hi, mid)

    lo, _ = lax.fori_loop(0, 64, body, (jnp.float32(-1.5), jnp.float32(1.0)))
    return lo


def _select(s2d, k):
    vk = _bisect(s2d, k)
    gt = s2d > vk
    eq = (s2d == vk).astype(jnp.float32)
    cnt_gt = jnp.sum(gt.astype(jnp.float32))
    need = jnp.float32(k) - cnt_gt
    nr, nc = s2d.shape
    ra = lax.broadcasted_iota(jnp.int32, (nc, nc), 0)
    rb = lax.broadcasted_iota(jnp.int32, (nc, nc), 1)
    m_incl = (ra <= rb).astype(jnp.float32)
    cum_in_row = jnp.dot(eq, m_incl, preferred_element_type=jnp.float32)
    ex_in_row = cum_in_row - eq
    rowtot = jnp.sum(eq, axis=1, keepdims=True)
    qa = lax.broadcasted_iota(jnp.int32, (nr, nr), 0)
    qb = lax.broadcasted_iota(jnp.int32, (nr, nr), 1)
    m_strict = (qb < qa).astype(jnp.float32)
    rowpre = jnp.dot(m_strict, rowtot, preferred_element_type=jnp.float32)
    rank = ex_in_row + rowpre
    return gt | ((eq > 0.0) & (rank < need))


def _sel_body(k, s_ref, w_ref, m_ref):
    s2d = jnp.reshape(s_ref[...], (_NPAD // 128, 128))
    sel = _select(s2d, k)
    w2d = jnp.where(sel, s2d, 0.0)
    w_ref[...] = jnp.reshape(w2d, (_NPAD, 1))
    m_ref[...] = jnp.reshape(sel.astype(jnp.float32), (_NPAD, 1))


def _sel(s, k):
    return pl.pallas_call(
        functools.partial(_sel_body, k),
        in_specs=[pl.BlockSpec((_NPAD, 1), lambda: (0, 0))],
        out_specs=[
            pl.BlockSpec((_NPAD, 1), lambda: (0, 0)),
            pl.BlockSpec((_NPAD, 1), lambda: (0, 0)),
        ],
        out_shape=[
            jax.ShapeDtypeStruct((_NPAD, 1), jnp.float32),
            jax.ShapeDtypeStruct((_NPAD, 1), jnp.float32),
        ],
        interpret=_INTERPRET,
    )(s)


def _scale_body(h_ref, w_ref, o_ref):
    o_ref[...] = h_ref[...] * w_ref[...]


def _scale(h, w):
    return pl.pallas_call(
        _scale_body,
        grid=(_NPAD // _BR,),
        in_specs=[
            pl.BlockSpec((_BR, 512), lambda i: (i, 0)),
            pl.BlockSpec((_BR, 1), lambda i: (i, 0)),
        ],
        out_specs=pl.BlockSpec((_BR, 512), lambda i: (i, 0)),
        out_shape=jax.ShapeDtypeStruct((_NPAD, 512), jnp.float32),
        interpret=_INTERPRET,
    )(h, w)


def _final_body(k, h_ref, s_ref, fcw_ref, fcb_ref, o_ref):
    s2d = jnp.reshape(s_ref[...], (_NPAD // 128, 128))
    sel = _select(s2d, k)
    w = jnp.reshape(jnp.where(sel, s2d, 0.0), (_NPAD, 1))
    pooled = jnp.sum(h_ref[...] * w, axis=0, keepdims=True) / jnp.float32(k)
    o_ref[...] = (jnp.dot(pooled, fcw_ref[...],
                          preferred_element_type=jnp.float32) + fcb_ref[...])


def _final(h, s, fcw, fcb, k):
    return pl.pallas_call(
        functools.partial(_final_body, k),
        in_specs=[
            pl.BlockSpec((_NPAD, 512), lambda: (0, 0)),
            pl.BlockSpec((_NPAD, 1), lambda: (0, 0)),
            pl.BlockSpec((512, 128), lambda: (0, 0)),
            pl.BlockSpec((1, 128), lambda: (0, 0)),
        ],
        out_specs=pl.BlockSpec((1, 128), lambda: (0, 0)),
        out_shape=jax.ShapeDtypeStruct((1, 128), jnp.float32),
        interpret=_INTERPRET,
    )(h, s, fcw, fcb.reshape(1, 128))


_NT = 32
_RT = _NPAD // _NT
_RH = _RT // 2
_CAP = 8192
_CAPP = _CAP + 32
_EB = 1600
_GB = 16

_sc_mesh = plsc.VectorSubcoreMesh(core_axis_name="c", subcore_axis_name="s")


def _bin_pass(src_hbm, dst_hbm, sbufs, ssems, listbuf, cbuf, tid):
    iota16 = lax.iota(jnp.int32, 16)
    base = tid * _RT
    ns = _E // _EB

    def stage_start(st, k):
        pltpu.async_copy(src_hbm.at[pl.ds(st * _EB, _EB)], sbufs[k][0],
                         ssems[k])
        pltpu.async_copy(dst_hbm.at[pl.ds(st * _EB, _EB)], sbufs[k][1],
                         ssems[k])

    def stage_wait(st, k):
        pltpu.make_async_copy(src_hbm.at[pl.ds(st * _EB, _EB)], sbufs[k][0],
                              ssems[k]).wait()
        pltpu.make_async_copy(dst_hbm.at[pl.ds(st * _EB, _EB)], sbufs[k][1],
                              ssems[k]).wait()

    stage_start(0, 0)
    stage_start(1, 1)

    def grp_body(g, cnts):
        for k in range(2):
            st = g * 2 + k
            stage_wait(st, k)
            ssrc, sdst = sbufs[k]

            def vec_body(v, cnts):
                c0, c1 = cnts
                dd = sdst[pl.ds(v * 16, 16)]
                ss = ssrc[pl.ds(v * 16, 16)]
                rel = dd - base
                pack = rel * 16384 + ss
                m0 = (rel >= 0) & (rel < _RH)
                cs0 = jnp.cumsum(m0.astype(jnp.int32))
                pos0 = c0 + cs0 - 1
                m0 = m0 & (pos0 < _CAP)
                plsc.store_scatter(listbuf, [pos0], pack, mask=m0)
                m1 = (rel >= _RH) & (rel < _RT)
                cs1 = jnp.cumsum(m1.astype(jnp.int32))
                pos1 = _CAPP + c1 + cs1 - 1
                m1 = m1 & (c1 + cs1 - 1 < _CAP)
                plsc.store_scatter(listbuf, [pos1], pack, mask=m1)
                return (c0 + cs0[15], c1 + cs1[15])

            cnts = lax.fori_loop(0, _EB // 16, vec_body, cnts)

            @pl.when(st + 2 < ns)
            def _():
                stage_start(st + 2, k)
        return cnts

    c0, c1 = lax.fori_loop(0, ns // 2, grp_body,
                           (jnp.int32(0), jnp.int32(0)))
    nbat0 = (c0 + 31) >> 5
    nbat1 = (c1 + 31) >> 5
    for half, (cnt, nbat) in enumerate(((c0, nbat0), (c1, nbat1))):
        for v in range(2):
            lpos = cnt + v * 16 + iota16
            m = lpos < nbat * 32
            pack = (half * _RH) * 16384 + 10224 + iota16
            plsc.store_scatter(listbuf, [half * _CAPP + lpos], pack, mask=m)
    cbuf[pl.ds(0, 16)] = jnp.where(iota16 == 0,
                                   jnp.full((16,), nbat0, jnp.int32),
                                   jnp.full((16,), nbat1, jnp.int32))
    return nbat0, nbat1


def _acc_pass(x_hbm, out_hbm, zeros_hbm, listbuf, acc, rowsbufs, gbufs, sems,
              nbat, tid, half):
    pltpu.sync_copy(zeros_hbm, acc)
    lbase = half * _CAPP
    rsub = half * _RH

    def prep(b, k):
        for v in range(_GB // 16):
            e16 = listbuf[pl.ds(lbase + b * _GB + v * 16, 16)]
            gbufs[k][pl.ds(v * 16, 16)] = e16 & 16383
        return pltpu.async_copy(x_hbm.at[gbufs[k]], rowsbufs[k], sems[k])

    @pl.when(nbat > 0)
    def _():
        prep(0, 0)

    @pl.when(nbat > 1)
    def _():
        prep(1, 1)

    def grp_body(g, carry):
        for k in range(2):
            b = g * 2 + k

            @pl.when(b < nbat)
            def _():
                pltpu.make_async_copy(x_hbm.at[gbufs[k]], rowsbufs[k],
                                      sems[k]).wait()
                rows = rowsbufs[k]
                rels = []
                for v in range(_GB // 16):
                    e16 = listbuf[pl.ds(lbase + b * _GB + v * 16, 16)]
                    rels.append(lax.shift_right_logical(e16, 14) - rsub)
                for j in range(_GB):
                    rel = rels[j // 16][j % 16]
                    for ci in range(512 // 16):
                        sl = pl.ds(ci * 16, 16)
                        plsc.addupdate(acc.at[rel, sl], rows[j, sl])

                @pl.when(b + 2 < nbat)
                def _():
                    prep(b + 2, k)
        return carry

    lax.fori_loop(0, (nbat + 1) >> 1, grp_body, 0)
    pltpu.sync_copy(acc, out_hbm.at[pl.ds(tid * _RT + rsub, _RH)])


@functools.partial(
    pl.kernel,
    out_type=[
        jax.ShapeDtypeStruct((_NPAD, 512), jnp.float32),
        jax.ShapeDtypeStruct((_NT, 2 * _CAPP), jnp.int32),
        jax.ShapeDtypeStruct((_NT, 16), jnp.int32),
    ],
    mesh=_sc_mesh,
    compiler_params=pltpu.CompilerParams(needs_layout_passes=False),
    scratch_types=[
        pltpu.VMEM((_EB,), jnp.int32),
        pltpu.VMEM((_EB,), jnp.int32),
        pltpu.VMEM((_EB,), jnp.int32),
        pltpu.VMEM((_EB,), jnp.int32),
        pltpu.VMEM((2 * _CAPP,), jnp.int32),
        pltpu.VMEM((16,), jnp.int32),
        pltpu.VMEM((_RH, 512), jnp.float32),
        pltpu.VMEM((_GB, 512), jnp.float32),
        pltpu.VMEM((_GB, 512), jnp.float32),
        pltpu.VMEM((_GB,), jnp.int32),
        pltpu.VMEM((_GB,), jnp.int32),
        pltpu.SemaphoreType.DMA,
        pltpu.SemaphoreType.DMA,
        pltpu.SemaphoreType.DMA,
        pltpu.SemaphoreType.DMA,
    ],
)
def _sc_agg_a(x_hbm, src_hbm, dst_hbm, zeros_hbm,
              agg_hbm, lists_hbm, counts_hbm,
              sa0, sa1, sb0, sb1, listbuf, cbuf, acc, rows0, rows1,
              g0, g1, s0, s1, t0, t1):
    tid = lax.axis_index("c") * 16 + lax.axis_index("s")
    nbat0, nbat1 = _bin_pass(src_hbm, dst_hbm,
                             ((sa0, sa1), (sb0, sb1)), (t0, t1),
                             listbuf, cbuf, tid)
    pltpu.sync_copy(cbuf, counts_hbm.at[tid])
    pltpu.sync_copy(listbuf, lists_hbm.at[tid])
    _acc_pass(x_hbm, agg_hbm, zeros_hbm, listbuf, acc, (rows0, rows1),
              (g0, g1), (s0, s1), nbat0, tid, 0)
    _acc_pass(x_hbm, agg_hbm, zeros_hbm, listbuf, acc, (rows0, rows1),
              (g0, g1), (s0, s1), nbat1, tid, 1)


@functools.partial(
    pl.kernel,
    out_type=jax.ShapeDtypeStruct((_NPAD, 512), jnp.float32),
    mesh=_sc_mesh,
    compiler_params=pltpu.CompilerParams(needs_layout_passes=False),
    scratch_types=[
        pltpu.VMEM((2 * _CAPP,), jnp.int32),
        pltpu.VMEM((16,), jnp.int32),
        pltpu.VMEM((_RH, 512), jnp.float32),
        pltpu.VMEM((_GB, 512), jnp.float32),
        pltpu.VMEM((_GB, 512), jnp.float32),
        pltpu.VMEM((_GB,), jnp.int32),
        pltpu.VMEM((_GB,), jnp.int32),
        pltpu.SemaphoreType.DMA,
        pltpu.SemaphoreType.DMA,
    ],
)
def _sc_agg_b(x_hbm, lists_hbm, counts_hbm, zeros_hbm, agg_hbm,
              listbuf, cbuf, acc, rows0, rows1, g0, g1, s0, s1):
    tid = lax.axis_index("c") * 16 + lax.axis_index("s")
    pltpu.sync_copy(lists_hbm.at[tid], listbuf)
    pltpu.sync_copy(counts_hbm.at[tid], cbuf)
    cb = cbuf[pl.ds(0, 16)]
    _acc_pass(x_hbm, agg_hbm, zeros_hbm, listbuf, acc, (rows0, rows1),
              (g0, g1), (s0, s1), cb[0], tid, 0)
    _acc_pass(x_hbm, agg_hbm, zeros_hbm, listbuf, acc, (rows0, rows1),
              (g0, g1), (s0, s1), cb[1], tid, 1)


def kernel(x, edge_index, batch, Wr1, Wn1, b1, pw1, Wr2, Wn2, b2, pw2, fcW, fcb):
    del batch
    src, dst = edge_index[0], edge_index[1]
    k1 = int(math.ceil(0.8 * _N))
    k2 = int(math.ceil(0.8 * k1))
    nrm1 = jnp.linalg.norm(pw1).reshape(1, 1)
    nrm2 = jnp.linalg.norm(pw2).reshape(1, 1)

    xp = jnp.pad(x, ((0, _NPAD - _N), (0, 0)))
    ones = jnp.ones((_NPAD, 1), jnp.float32)
    zeros = jnp.zeros((_RH, 512), jnp.float32)

    v1, u1 = _mmw(xp, Wn1, Wr1)
    agg1, lists, counts = _sc_agg_a(v1, src, dst, zeros)
    h1, s1 = _mmb(u1, agg1, b1, pw1, nrm1, ones)
    w1, m1 = _sel(s1, k1)
    x2 = _scale(h1, w1)
    v2, u2 = _mmw(x2, Wn2, Wr2)
    agg2 = _sc_agg_b(v2, lists, counts, zeros)
    h2, s2 = _mmb(u2, agg2, b2, pw2, nrm2, m1)
    return _final(h2, s2, fcW, fcb, k2)
